# baseline (speedup 1.0000x reference)
import functools
import math

import jax
import jax.numpy as jnp
from jax import lax
from jax.experimental import pallas as pl
from jax.experimental.pallas import tpu as pltpu
from jax.experimental.pallas import tpu_sc as plsc

N = 10000
E = 320000
D = 128
NAGG = 10112
NDEG = 10240
DROWS = NDEG // D
NC, NS = 2, 16
NW = NC * NS
EPW = E // NW
CH = 80
NCHUNK = EPW // CH
QCH = 4 * CH
NQ = 31
RPT = NAGG // NS
EPS = 1e-7
CLIP = math.pi / 2 - 1e-3
BLK = 2048
NBLK = -(-N // BLK)
DBLK = BLK // D


def _logmap0(x):
    nrm = jnp.maximum(jnp.sqrt(jnp.sum(x * x, axis=1, keepdims=True)), EPS)
    return jnp.arctan2(nrm, jnp.ones_like(nrm)) * x / nrm


def _dense_body(x_ref, w_ref, b_ref, o_ref):
    xt = _logmap0(x_ref[...])
    o_ref[...] = (
        jnp.dot(xt, w_ref[...], preferred_element_type=jnp.float32) + b_ref[...]
    )


def _deg_expand(dgrid):
    row = lax.broadcasted_iota(jnp.int32, (BLK, DBLK), 0)
    g = lax.broadcasted_iota(jnp.int32, (BLK, DBLK), 1)
    onehot = (g == lax.shift_right_logical(row, 7)).astype(jnp.float32)
    sel = jnp.dot(onehot, dgrid, preferred_element_type=jnp.float32)
    rowl = lax.broadcasted_iota(jnp.int32, (BLK, D), 0)
    lane = lax.broadcasted_iota(jnp.int32, (BLK, D), 1)
    lmask = (lane == (rowl & 127)).astype(jnp.float32)
    return jnp.sum(sel * lmask, axis=1, keepdims=True)


def _combine(h_ref, a0_ref, a1_ref, d0_ref, d1_ref):
    agg = a0_ref[...] + a1_ref[...]
    deg = _deg_expand(d0_ref[...] + d1_ref[...])
    t = jnp.maximum((h_ref[...] + agg) / (deg + 1.0), 0.0)
    nt = jnp.maximum(jnp.sqrt(jnp.sum(t * t, axis=1, keepdims=True)), EPS)
    return t, nt


def _combine_dense_body(h_ref, a0_ref, a1_ref, d0_ref, d1_ref,
                        w_ref, bias_ref, o_ref):
    t, nt = _combine(h_ref, a0_ref, a1_ref, d0_ref, d1_ref)
    xt = t * (jnp.minimum(nt, CLIP) / nt)
    o_ref[...] = (
        jnp.dot(xt, w_ref[...], preferred_element_type=jnp.float32) + bias_ref[...]
    )


def _combine_out_body(h_ref, a0_ref, a1_ref, d0_ref, d1_ref, o_ref):
    t, nt = _combine(h_ref, a0_ref, a1_ref, d0_ref, d1_ref)
    o_ref[...] = jnp.tan(jnp.minimum(nt, CLIP)) * t / nt


def _segsum_body(layer,
                 h_hbm, adj_hbm,
                 agg0_hbm, agg1_hbm, deg0_hbm, deg1_hbm,
                 sq0, sq1, dst_flat, dsta, dstb, rows_a, rows_b,
                 zero_v, iota_v, deg_l, agg_sh, deg_sh,
                 sem_s0, sem_s1, sem_d, sem_a, sem_b, sem_ca, sem_cb):
    c = lax.axis_index("c")
    s = lax.axis_index("s")
    wid = s * NC + c
    e0 = wid * EPW

    zrow = jnp.zeros((16,), jnp.float32)

    src0 = 2 * layer * E + e0
    dst0 = (2 * layer + 1) * E + e0

    ld_d = pltpu.async_copy(adj_hbm.at[pl.ds(dst0, EPW)], dst_flat, sem_d)
    pltpu.async_copy(adj_hbm.at[pl.ds(src0, QCH)], sq0, sem_s0)
    pltpu.async_copy(adj_hbm.at[pl.ds(src0 + QCH, QCH)], sq1, sem_s1)

    for r in range(8):
        for j in range(D // 16):
            zero_v[r, pl.ds(j * 16, 16)] = zrow
    for j in range(DROWS // 16):
        iota_v[pl.ds(j * 16, 16)] = lax.iota(jnp.int32, 16) + j * 16

    def zagg(i, carry):
        zs = [pltpu.async_copy(
            zero_v, agg_sh.at[pl.ds(s * RPT + (i * 8 + k) * 8, 8)], sem_a)
            for k in range(8)]
        for z in zs:
            z.wait()
        return carry
    lax.fori_loop(0, RPT // 64, zagg, 0)
    for k in range(RPT // 8 - (RPT // 64) * 8):
        pltpu.sync_copy(zero_v,
                        agg_sh.at[pl.ds(s * RPT + ((RPT // 64) * 64 + k * 8), 8)])

    def zdegl(i, carry):
        for j in range(16):
            deg_l[pl.ds(i * 256 + j * 16, 16)] = zrow
        return carry
    lax.fori_loop(0, NDEG // 256, zdegl, 0)

    @pl.when(s == 0)
    def _():
        for k in range(DROWS // 8):
            pltpu.sync_copy(zero_v, deg_sh.at[pl.ds(k * 8, 8)])
    ld_d.wait()
    plsc.subcore_barrier()

    ones16 = jnp.ones((16,), jnp.float32)

    def drain(rows, sem):
        pltpu.make_async_copy(h_hbm.at[pl.ds(0, CH)], rows, sem).wait()

    def do_chunk(ch, sq, qoff, rows, dst2, sem_g, sem_c, may_be_first, i=None):
        if may_be_first:
            @pl.when(i > 0)
            def _():
                drain(rows, sem_c)
        else:
            drain(rows, sem_c)
        g = pltpu.async_copy(h_hbm.at[sq.at[pl.ds(qoff * CH, CH)]], rows,
                             sem_g)
        for j in range(CH // 16):
            dvec = dst_flat[pl.ds(ch * CH + j * 16, 16)]
            dst2[0, pl.ds(j * 16, 16)] = dvec
            plsc.addupdate_scatter(deg_l, [dvec], ones16)
        return g

    def do_quad(q, sq, may_be_first, i=None):
        qc = q * 4
        for p in range(2):
            ga = do_chunk(qc + 2 * p, sq, 2 * p, rows_a, dsta,
                          sem_a, sem_ca, may_be_first and p == 0, i)
            gb = do_chunk(qc + 2 * p + 1, sq, 2 * p + 1, rows_b, dstb,
                          sem_b, sem_cb, may_be_first and p == 0, i)
            ga.wait()
            pltpu.async_copy(rows_a, agg_sh.at[dsta.at[0]], sem_ca, add=True)
            gb.wait()
            pltpu.async_copy(rows_b, agg_sh.at[dstb.at[0]], sem_cb, add=True)

    def qiter(i, carry):
        q0 = 2 * i
        pltpu.make_async_copy(adj_hbm.at[pl.ds(src0, QCH)], sq0,
                              sem_s0).wait()
        do_quad(q0, sq0, True, i)
        pltpu.async_copy(adj_hbm.at[pl.ds(src0 + (q0 + 2) * QCH, QCH)],
                         sq0, sem_s0)
        pltpu.make_async_copy(adj_hbm.at[pl.ds(src0, QCH)], sq1,
                              sem_s1).wait()
        do_quad(q0 + 1, sq1, False)

        @pl.when(i < NQ // 2 - 1)
        def _():
            pltpu.async_copy(
                adj_hbm.at[pl.ds(src0 + (q0 + 3) * QCH, QCH)],
                sq1, sem_s1)
        return carry
    lax.fori_loop(0, NQ // 2, qiter, 0)

    pltpu.make_async_copy(adj_hbm.at[pl.ds(src0, QCH)], sq0,
                          sem_s0).wait()
    do_quad(NQ - 1, sq0, False)
    drain(rows_a, sem_ca)
    ct = NCHUNK - 1
    pltpu.sync_copy(adj_hbm.at[pl.ds(src0 + ct * CH, CH)],
                    sq0.at[pl.ds(0, CH)])
    pltpu.async_copy(h_hbm.at[sq0.at[pl.ds(0, CH)]], rows_a, sem_a).wait()
    for j in range(CH // 16):
        dvec = dst_flat[pl.ds(ct * CH + j * 16, 16)]
        dsta[0, pl.ds(j * 16, 16)] = dvec
        plsc.addupdate_scatter(deg_l, [dvec], ones16)
    pltpu.sync_copy(rows_a, agg_sh.at[dsta.at[0]], add=True)
    drain(rows_b, sem_cb)

    def dconv(r, carry):
        for j in range(D // 16):
            rows_a[r, pl.ds(j * 16, 16)] = deg_l[pl.ds(r * D + j * 16, 16)]
        return carry
    lax.fori_loop(0, DROWS, dconv, 0)
    pltpu.sync_copy(rows_a, deg_sh.at[iota_v], add=True)
    plsc.subcore_barrier()

    stripe = pl.ds(s * RPT, RPT)

    @pl.when(c == 0)
    def _():
        pltpu.sync_copy(agg_sh.at[stripe], agg0_hbm.at[stripe])

    @pl.when(c == 1)
    def _():
        pltpu.sync_copy(agg_sh.at[stripe], agg1_hbm.at[stripe])

    @pl.when((c == 0) & (s == 0))
    def _():
        pltpu.sync_copy(deg_sh, deg0_hbm)

    @pl.when((c == 1) & (s == 0))
    def _():
        pltpu.sync_copy(deg_sh, deg1_hbm)


def _segsum(h, adj, layer):
    agg_t = jax.ShapeDtypeStruct((NAGG, D), jnp.float32)
    deg_t = jax.ShapeDtypeStruct((DROWS, D), jnp.float32)
    kern = pl.kernel(
        functools.partial(_segsum_body, layer),
        out_type=(agg_t, agg_t, deg_t, deg_t),
        mesh=plsc.VectorSubcoreMesh(core_axis_name="c", subcore_axis_name="s"),
        compiler_params=pltpu.CompilerParams(needs_layout_passes=False),
        scratch_types=[
            pltpu.VMEM((QCH,), jnp.int32),
            pltpu.VMEM((QCH,), jnp.int32),
            pltpu.VMEM((EPW,), jnp.int32),
            pltpu.VMEM((1, CH), jnp.int32),
            pltpu.VMEM((1, CH), jnp.int32),
            pltpu.VMEM((CH, D), jnp.float32),
            pltpu.VMEM((CH, D), jnp.float32),
            pltpu.VMEM((8, D), jnp.float32),
            pltpu.VMEM((DROWS,), jnp.int32),
            pltpu.VMEM((NDEG,), jnp.float32),
            pltpu.VMEM_SHARED((NAGG, D), jnp.float32),
            pltpu.VMEM_SHARED((DROWS, D), jnp.float32),
        ] + [pltpu.SemaphoreType.DMA] * 7,
    )
    return kern(h, adj.reshape(4 * E))


def _dense(x, W, b):
    return pl.pallas_call(
        _dense_body,
        grid=(NBLK,),
        in_specs=[
            pl.BlockSpec((BLK, D), lambda i: (i, 0)),
            pl.BlockSpec((D, D), lambda i: (0, 0)),
            pl.BlockSpec((1, D), lambda i: (0, 0)),
        ],
        out_specs=pl.BlockSpec((BLK, D), lambda i: (i, 0)),
        out_shape=jax.ShapeDtypeStruct((N, D), jnp.float32),
    )(x, W, b.reshape(1, D))


_node_specs = [
    pl.BlockSpec((BLK, D), lambda i: (i, 0)),
    pl.BlockSpec((BLK, D), lambda i: (i, 0)),
    pl.BlockSpec((BLK, D), lambda i: (i, 0)),
    pl.BlockSpec((DBLK, D), lambda i: (i, 0)),
    pl.BlockSpec((DBLK, D), lambda i: (i, 0)),
]


def _combine_dense(h, a0, a1, d0, d1, W, b):
    return pl.pallas_call(
        _combine_dense_body,
        grid=(NBLK,),
        in_specs=_node_specs + [
            pl.BlockSpec((D, D), lambda i: (0, 0)),
            pl.BlockSpec((1, D), lambda i: (0, 0)),
        ],
        out_specs=pl.BlockSpec((BLK, D), lambda i: (i, 0)),
        out_shape=jax.ShapeDtypeStruct((N, D), jnp.float32),
    )(h, a0, a1, d0, d1, W, b.reshape(1, D))


def _combine_out(h, a0, a1, d0, d1):
    return pl.pallas_call(
        _combine_out_body,
        grid=(NBLK,),
        in_specs=_node_specs,
        out_specs=pl.BlockSpec((BLK, D), lambda i: (i, 0)),
        out_shape=jax.ShapeDtypeStruct((N, D), jnp.float32),
    )(h, a0, a1, d0, d1)


def kernel(x, adj, W1, b1, W2, b2):
    adj = adj.astype(jnp.int32)
    h1 = _dense(x, W1, b1)
    a10, a11, d10, d11 = _segsum(h1, adj, 0)
    h2 = _combine_dense(h1, a10, a11, d10, d11, W2, b2)
    a20, a21, d20, d21 = _segsum(h2, adj, 1)
    return _combine_out(h2, a20, a21, d20, d21)
